# Initial kernel scaffold; baseline (speedup 1.0000x reference)
import jax
import jax.numpy as jnp
from jax.experimental import pallas as pl


def kernel(loc_data, conf_data, prior_data):
    raise NotImplementedError("write your pallas kernel here")


# speedup vs baseline: 311.8684x; 311.8684x over previous
import functools

import jax
import jax.numpy as jnp
from jax import lax
from jax.experimental import pallas as pl
from jax.experimental.pallas import tpu as pltpu
from jax.experimental.pallas import tpu_sc as plsc

_NUM_CLASSES = 21
_TOP_K = 200
_CONF = 0.01
_NMS_T = 0.45
_N = 20000
_NV = _N // 16
_NC, _NS = 2, 16
_NW = _NC * _NS
_TASKS = 40
_CAND = 208
_CV = _CAND // 16


def _do_task(task, conf_hbm, feat_hbm, out_hbm, scores_v, keys_v, hist_v,
             ring_v, cand_v, idx_v, feats_v, sc_v, ms_v, x1_v, y1_v, x2_v,
             y2_v, ar_v, outbuf_v, sem, lane):
    ones16 = jnp.ones((16,), jnp.int32)
    zero16 = jnp.zeros((16,), jnp.int32)
    neg_inf = jnp.full((16,), -jnp.inf, jnp.float32)

    pltpu.sync_copy(conf_hbm.at[task], scores_v)

    prefix = jnp.int32(0)
    need = jnp.int32(_TOP_K)
    for rnd in range(4):
        shift = 24 - 8 * rnd

        def zero_hist(b, _):
            hist_v[pl.ds(b * 16, 16)] = zero16
            return 0

        lax.fori_loop(0, 256, zero_hist, 0)

        if rnd == 0:
            def hist_body(j, _):
                s = scores_v[pl.ds(j * 16, 16)]
                key = jnp.where(s > _CONF, plsc.bitcast(s, jnp.int32) + 1, 0)
                keys_v[pl.ds(j * 16, 16)] = key
                byte = lax.shift_right_logical(key, shift)
                plsc.addupdate_scatter(hist_v, [byte * 16 + lane], ones16)
                return 0
        else:
            def hist_body(j, _, shift=shift, prefix=prefix):
                key = keys_v[pl.ds(j * 16, 16)]
                m = lax.shift_right_logical(key, shift + 8) == prefix
                byte = lax.shift_right_logical(key, shift) & 0xFF
                plsc.addupdate_scatter(hist_v, [byte * 16 + lane], ones16,
                                       mask=m)
                return 0

        lax.fori_loop(0, _NV, hist_body, 0)

        def scan_body(k, carry, need=need):
            cum, bsel, cnt_above = carry
            b = 255 - k
            c = jnp.sum(hist_v[pl.ds(b * 16, 16)])
            hit = (bsel < 0) & (cum + c >= need)
            bsel = jnp.where(hit, b, bsel)
            cnt_above = jnp.where(hit, cum, cnt_above)
            return cum + c, bsel, cnt_above

        _, bsel, cnt_above = lax.fori_loop(
            0, 256, scan_body, (jnp.int32(0), jnp.int32(-1), jnp.int32(0)))
        need = need - cnt_above
        prefix = prefix * 256 + bsel
    kth = prefix

    def zero_cand(c, _):
        cand_v[pl.ds(c * 16, 16)] = zero16
        return 0

    lax.fori_loop(0, _CV, zero_cand, 0)

    def comp_body(j, carry):
        gt_base, eq_base = carry
        key = keys_v[pl.ds(j * 16, 16)]
        gidx = j * 16 + lane
        m_gt = key > kth
        pos = jnp.maximum(gt_base + plsc.cumsum(jnp.where(m_gt, 1, 0)) - 1, 0)
        plsc.store_scatter(cand_v, [pos], gidx, mask=m_gt)
        gt_base = gt_base + plsc.all_reduce_population_count(m_gt)
        m_eq = key == kth
        pos_e = (eq_base + plsc.cumsum(jnp.where(m_eq, 1, 0)) - 1) & 255
        plsc.store_scatter(ring_v, [pos_e], gidx, mask=m_eq)
        eq_base = eq_base + plsc.all_reduce_population_count(m_eq)
        return gt_base, eq_base

    gt_base, eq_base = lax.fori_loop(0, _NV, comp_body, (zero16, zero16))
    n_gt = jnp.max(gt_base)
    n_eq = jnp.max(eq_base)
    need4 = _TOP_K - n_gt
    for c in range(_CV):
        offs = c * 16 + lane
        selm = offs < need4
        src = (n_eq - need4 + offs) & 255
        val = plsc.load_gather(ring_v, [src], mask=selm)
        dst = jnp.minimum(n_gt + offs, _CAND - 1)
        plsc.store_scatter(cand_v, [dst], val, mask=selm)

    ioff = jnp.where(task >= 20, _N, 0)
    for c in range(_CV):
        base = pl.ds(c * 16, 16)
        slot = c * 16 + lane
        ci = cand_v[base]
        valid_slot = slot < _TOP_K
        ci = jnp.where(valid_slot, ci, 0)
        scv = plsc.load_gather(scores_v, [ci])
        active = valid_slot & (scv > _CONF)
        sc_v[base] = scv
        ms_v[base] = jnp.where(active, scv, neg_inf)
        idx_v[base] = ci + ioff

    pltpu.async_copy(feat_hbm.at[idx_v.at[pl.ds(0, 104)]],
                     feats_v.at[pl.ds(0, 104), :], sem).wait()
    pltpu.async_copy(feat_hbm.at[idx_v.at[pl.ds(104, 104)]],
                     feats_v.at[pl.ds(104, 104), :], sem).wait()

    for c in range(_CV):
        base = pl.ds(c * 16, 16)
        row = c * 16 + lane

        def col(k):
            return plsc.load_gather(feats_v,
                                    [row, jnp.full((16,), k, jnp.int32)])

        lx, ly, lw, lh = col(0), col(1), col(2), col(3)
        px, py, pw, ph = col(4), col(5), col(6), col(7)
        x = px + (lx * 0.1) * pw
        y = py + (ly * 0.1) * ph
        w = pw * jnp.exp(lw * 0.2)
        h = ph * jnp.exp(lh * 0.2)
        x1 = x - w * 0.5
        y1 = y - h * 0.5
        x2 = w + x1
        y2 = h + y1
        x1_v[base] = x1
        y1_v[base] = y1
        x2_v[base] = x2
        y2_v[base] = y2
        ar_v[base] = (x2 - x1) * (y2 - y1)

    def nms_body(t, _):
        def p1(c, acc):
            return jnp.maximum(acc, ms_v[pl.ds(c * 16, 16)])

        acc = lax.fori_loop(0, _CV, p1, neg_inf)
        m = jnp.max(acc)
        any_active = m > -jnp.inf

        def p2(c, best):
            msv = ms_v[pl.ds(c * 16, 16)]
            slot = c * 16 + lane
            return jnp.maximum(best, jnp.where(msv == m, slot, -1))

        bestv = lax.fori_loop(0, _CV, p2, jnp.full((16,), -1, jnp.int32))
        sel = jnp.maximum(jnp.max(bestv), 0)
        sels = jnp.full((16,), 0, jnp.int32) + sel
        sx1 = plsc.load_gather(x1_v, [sels])
        sy1 = plsc.load_gather(y1_v, [sels])
        sx2 = plsc.load_gather(x2_v, [sels])
        sy2 = plsc.load_gather(y2_v, [sels])
        sar = plsc.load_gather(ar_v, [sels])
        ssc = plsc.load_gather(sc_v, [sels])

        def p3(c, _):
            base = pl.ds(c * 16, 16)
            xx1 = jnp.maximum(x1_v[base], sx1)
            yy1 = jnp.maximum(y1_v[base], sy1)
            xx2 = jnp.minimum(x2_v[base], sx2)
            yy2 = jnp.minimum(y2_v[base], sy2)
            inter = (jnp.maximum(xx2 - xx1, 0.0)
                     * jnp.maximum(yy2 - yy1, 0.0))
            union = ar_v[base] + sar - inter
            iou = inter / union
            msv = ms_v[base]
            new_ms = jnp.where(iou <= _NMS_T, msv, neg_inf)
            slot = c * 16 + lane
            new_ms = jnp.where(slot == sel, neg_inf, new_ms)
            ms_v[base] = jnp.where(any_active, new_ms, msv)
            return 0

        lax.fori_loop(0, _CV, p3, 0)

        ovals = jnp.where(lane == 0, ssc,
                          jnp.where(lane == 1, sx1,
                                    jnp.where(lane == 2, sy1,
                                              jnp.where(lane == 3, sx2,
                                                        sy2))))
        ovals = jnp.where(any_active, ovals, 0.0)
        plsc.store_scatter(outbuf_v, [t * 5 + lane], ovals, mask=lane < 5)
        return 0

    lax.fori_loop(0, _TOP_K, nms_body, 0)

    pltpu.sync_copy(outbuf_v, out_hbm.at[task])


@functools.partial(
    pl.kernel,
    out_type=jax.ShapeDtypeStruct((_TASKS, _TOP_K * 5), jnp.float32),
    mesh=plsc.VectorSubcoreMesh(core_axis_name="c", subcore_axis_name="s"),
    scratch_types=[
        pltpu.VMEM((_N,), jnp.float32),
        pltpu.VMEM((_N,), jnp.int32),
        pltpu.VMEM((4096,), jnp.int32),
        pltpu.VMEM((256,), jnp.int32),
        pltpu.VMEM((_CAND,), jnp.int32),
        pltpu.VMEM((_CAND,), jnp.int32),
        pltpu.VMEM((_CAND, 8), jnp.float32),
        pltpu.VMEM((_CAND,), jnp.float32),
        pltpu.VMEM((_CAND,), jnp.float32),
        pltpu.VMEM((_CAND,), jnp.float32),
        pltpu.VMEM((_CAND,), jnp.float32),
        pltpu.VMEM((_CAND,), jnp.float32),
        pltpu.VMEM((_CAND,), jnp.float32),
        pltpu.VMEM((_CAND,), jnp.float32),
        pltpu.VMEM((_TOP_K * 5,), jnp.float32),
        pltpu.SemaphoreType.DMA,
    ],
    compiler_params=pltpu.CompilerParams(needs_layout_passes=False,
                                         use_tc_tiling_on_sc=False),
)
def _sc_detect(conf_hbm, feat_hbm, out_hbm, scores_v, keys_v, hist_v, ring_v,
               cand_v, idx_v, feats_v, sc_v, ms_v, x1_v, y1_v, x2_v, y2_v,
               ar_v, outbuf_v, sem):
    wid = lax.axis_index("s") * _NC + lax.axis_index("c")
    lane = lax.iota(jnp.int32, 16)
    args = (conf_hbm, feat_hbm, out_hbm, scores_v, keys_v, hist_v, ring_v,
            cand_v, idx_v, feats_v, sc_v, ms_v, x1_v, y1_v, x2_v, y2_v, ar_v,
            outbuf_v, sem, lane)
    _do_task(wid, *args)

    @pl.when(wid + _NW < _TASKS)
    def _():
        _do_task(wid + _NW, *args)


def kernel(loc_data, conf_data, prior_data):
    num = loc_data.shape[0]
    conf_rows = jnp.transpose(conf_data[:, :, 1:], (0, 2, 1)).reshape(
        num * (_NUM_CLASSES - 1), _N)
    feat = jnp.concatenate(
        [loc_data.reshape(num * _N, 4),
         jnp.broadcast_to(prior_data, (num, _N, 4)).reshape(num * _N, 4)],
        axis=1)
    out40 = _sc_detect(conf_rows, feat)
    out = out40.reshape(num, _NUM_CLASSES - 1, _TOP_K, 5)
    zeros0 = jnp.zeros((num, 1, _TOP_K, 5), jnp.float32)
    return jnp.concatenate([zeros0, out], axis=1)


# speedup vs baseline: 477.7989x; 1.5321x over previous
import functools

import jax
import jax.numpy as jnp
from jax import lax
from jax.experimental import pallas as pl
from jax.experimental.pallas import tpu as pltpu
from jax.experimental.pallas import tpu_sc as plsc

_NUM_CLASSES = 21
_TOP_K = 200
_CONF = 0.01
_NMS_T = 0.45
_N = 20000
_NV = _N // 16
_NC, _NS = 2, 16
_NW = _NC * _NS
_TASKS = 40
_CAND = 208
_CV = _CAND // 16
_UNROLL = 5


def _do_task(task, conf_hbm, feat_hbm, out_hbm, scores_v, keys_v, hist_v,
             ring_v, cand_v, idx_v, feats_v, sc_v, x1_v, y1_v, x2_v,
             y2_v, ar_v, outbuf_v, sem, lane):
    ones16 = jnp.ones((16,), jnp.int32)
    zero16 = jnp.zeros((16,), jnp.int32)
    neg_inf = jnp.full((16,), -jnp.inf, jnp.float32)

    pltpu.sync_copy(conf_hbm.at[task], scores_v)

    prefix = jnp.int32(0)
    need = jnp.int32(_TOP_K)
    for rnd in range(4):
        shift = 24 - 8 * rnd

        if rnd == 0:
            def hist_body(j, _):
                for u in range(_UNROLL):
                    base = pl.ds((j * _UNROLL + u) * 16, 16)
                    s = scores_v[base]
                    key = jnp.where(s > _CONF,
                                    plsc.bitcast(s, jnp.int32) + 1, 0)
                    keys_v[base] = key
                    byte = lax.shift_right_logical(key, shift)
                    plsc.addupdate_scatter(hist_v, [byte * 16 + lane], ones16)
                return 0
        else:
            def hist_body(j, _, shift=shift, prefix=prefix):
                for u in range(_UNROLL):
                    base = pl.ds((j * _UNROLL + u) * 16, 16)
                    key = keys_v[base]
                    m = lax.shift_right_logical(key, shift + 8) == prefix
                    byte = lax.shift_right_logical(key, shift) & 0xFF
                    plsc.addupdate_scatter(hist_v, [byte * 16 + lane],
                                           ones16, mask=m)
                return 0

        lax.fori_loop(0, _NV // _UNROLL, hist_body, 0)

        def scan_body(k, carry, need=need):
            cum, bsel, cnt_above = carry
            b = 255 - k
            row = pl.ds(b * 16, 16)
            c = jnp.sum(hist_v[row])
            hist_v[row] = zero16
            hit = (bsel < 0) & (cum + c >= need)
            bsel = jnp.where(hit, b, bsel)
            cnt_above = jnp.where(hit, cum, cnt_above)
            return cum + c, bsel, cnt_above

        _, bsel, cnt_above = lax.fori_loop(
            0, 256, scan_body, (jnp.int32(0), jnp.int32(-1), jnp.int32(0)))
        need = need - cnt_above
        prefix = prefix * 256 + bsel
    kth = prefix

    for c in range(_CV):
        cand_v[pl.ds(c * 16, 16)] = zero16

    def comp_body(j, carry):
        gt_base, eq_base = carry
        for u in range(_UNROLL):
            jj = j * _UNROLL + u
            key = keys_v[pl.ds(jj * 16, 16)]
            gidx = jj * 16 + lane
            m_gt = key > kth
            pos = jnp.maximum(
                gt_base + plsc.cumsum(jnp.where(m_gt, 1, 0)) - 1, 0)
            plsc.store_scatter(cand_v, [pos], gidx, mask=m_gt)
            gt_base = gt_base + plsc.all_reduce_population_count(m_gt)
            m_eq = key == kth
            pos_e = (eq_base + plsc.cumsum(jnp.where(m_eq, 1, 0)) - 1) & 255
            plsc.store_scatter(ring_v, [pos_e], gidx, mask=m_eq)
            eq_base = eq_base + plsc.all_reduce_population_count(m_eq)
        return gt_base, eq_base

    gt_base, eq_base = lax.fori_loop(0, _NV // _UNROLL, comp_body,
                                     (zero16, zero16))
    n_gt = jnp.max(gt_base)
    n_eq = jnp.max(eq_base)
    need4 = _TOP_K - n_gt
    for c in range(_CV):
        offs = c * 16 + lane
        selm = offs < need4
        src = (n_eq - need4 + offs) & 255
        val = plsc.load_gather(ring_v, [src], mask=selm)
        dst = jnp.minimum(n_gt + offs, _CAND - 1)
        plsc.store_scatter(cand_v, [dst], val, mask=selm)

    ioff = jnp.where(task >= 20, _N, 0)
    ms = []
    for c in range(_CV):
        base = pl.ds(c * 16, 16)
        slot = c * 16 + lane
        ci = cand_v[base]
        valid_slot = slot < _TOP_K
        ci = jnp.where(valid_slot, ci, 0)
        scv = plsc.load_gather(scores_v, [ci])
        active = valid_slot & (scv > _CONF)
        sc_v[base] = scv
        ms.append(jnp.where(active, scv, neg_inf))
        idx_v[base] = ci + ioff

    pltpu.async_copy(feat_hbm.at[idx_v.at[pl.ds(0, 104)]],
                     feats_v.at[pl.ds(0, 104), :], sem).wait()
    pltpu.async_copy(feat_hbm.at[idx_v.at[pl.ds(104, 104)]],
                     feats_v.at[pl.ds(104, 104), :], sem).wait()

    for c in range(_CV):
        base = pl.ds(c * 16, 16)
        row = c * 16 + lane

        def col(k):
            return plsc.load_gather(feats_v,
                                    [row, jnp.full((16,), k, jnp.int32)])

        lx, ly, lw, lh = col(0), col(1), col(2), col(3)
        px, py, pw, ph = col(4), col(5), col(6), col(7)
        x = px + (lx * 0.1) * pw
        y = py + (ly * 0.1) * ph
        w = pw * jnp.exp(lw * 0.2)
        h = ph * jnp.exp(lh * 0.2)
        x1 = x - w * 0.5
        y1 = y - h * 0.5
        x2 = w + x1
        y2 = h + y1
        x1_v[base] = x1
        y1_v[base] = y1
        x2_v[base] = x2
        y2_v[base] = y2
        ar_v[base] = (x2 - x1) * (y2 - y1)

    bestval = neg_inf
    bestslot = jnp.full((16,), -1, jnp.int32)
    for c in range(_CV):
        slotv = c * 16 + lane
        upd = ms[c] >= bestval
        bestslot = jnp.where(upd, slotv, bestslot)
        bestval = jnp.maximum(bestval, ms[c])
    m0 = jnp.max(bestval)
    sel0 = jnp.maximum(
        jnp.max(jnp.where(bestval == m0, bestslot, -1)), 0)

    def nms_body(t, carry):
        m = carry[0]
        sel = carry[1]
        ms = list(carry[2:])
        any_active = m > -jnp.inf
        sels = zero16 + sel
        sx1 = plsc.load_gather(x1_v, [sels])
        sy1 = plsc.load_gather(y1_v, [sels])
        sx2 = plsc.load_gather(x2_v, [sels])
        sy2 = plsc.load_gather(y2_v, [sels])
        sar = plsc.load_gather(ar_v, [sels])
        ssc = plsc.load_gather(sc_v, [sels])

        bestval = neg_inf
        bestslot = jnp.full((16,), -1, jnp.int32)
        new_ms = []
        for c in range(_CV):
            base = pl.ds(c * 16, 16)
            slotv = c * 16 + lane
            xx1 = jnp.maximum(x1_v[base], sx1)
            yy1 = jnp.maximum(y1_v[base], sy1)
            xx2 = jnp.minimum(x2_v[base], sx2)
            yy2 = jnp.minimum(y2_v[base], sy2)
            inter = (jnp.maximum(xx2 - xx1, 0.0)
                     * jnp.maximum(yy2 - yy1, 0.0))
            union = ar_v[base] + sar - inter
            iou = inter / union
            nm = jnp.where(iou <= _NMS_T, ms[c], neg_inf)
            nm = jnp.where(slotv == sel, neg_inf, nm)
            nm = jnp.where(any_active, nm, ms[c])
            upd = nm >= bestval
            bestslot = jnp.where(upd, slotv, bestslot)
            bestval = jnp.maximum(bestval, nm)
            new_ms.append(nm)
        m_next = jnp.max(bestval)
        sel_next = jnp.maximum(
            jnp.max(jnp.where(bestval == m_next, bestslot, -1)), 0)

        ovals = jnp.where(lane == 0, ssc,
                          jnp.where(lane == 1, sx1,
                                    jnp.where(lane == 2, sy1,
                                              jnp.where(lane == 3, sx2,
                                                        sy2))))
        ovals = jnp.where(any_active, ovals, 0.0)
        plsc.store_scatter(outbuf_v, [t * 5 + lane], ovals, mask=lane < 5)
        return (m_next, sel_next, *new_ms)

    lax.fori_loop(0, _TOP_K, nms_body, (m0, sel0, *ms))

    pltpu.sync_copy(outbuf_v, out_hbm.at[task])


@functools.partial(
    pl.kernel,
    out_type=jax.ShapeDtypeStruct((_TASKS, _TOP_K * 5), jnp.float32),
    mesh=plsc.VectorSubcoreMesh(core_axis_name="c", subcore_axis_name="s"),
    scratch_types=[
        pltpu.VMEM((_N,), jnp.float32),
        pltpu.VMEM((_N,), jnp.int32),
        pltpu.VMEM((4096,), jnp.int32),
        pltpu.VMEM((256,), jnp.int32),
        pltpu.VMEM((_CAND,), jnp.int32),
        pltpu.VMEM((_CAND,), jnp.int32),
        pltpu.VMEM((_CAND, 8), jnp.float32),
        pltpu.VMEM((_CAND,), jnp.float32),
        pltpu.VMEM((_CAND,), jnp.float32),
        pltpu.VMEM((_CAND,), jnp.float32),
        pltpu.VMEM((_CAND,), jnp.float32),
        pltpu.VMEM((_CAND,), jnp.float32),
        pltpu.VMEM((_CAND,), jnp.float32),
        pltpu.VMEM((_TOP_K * 5,), jnp.float32),
        pltpu.SemaphoreType.DMA,
    ],
    compiler_params=pltpu.CompilerParams(needs_layout_passes=False,
                                         use_tc_tiling_on_sc=False),
)
def _sc_detect(conf_hbm, feat_hbm, out_hbm, scores_v, keys_v, hist_v, ring_v,
               cand_v, idx_v, feats_v, sc_v, x1_v, y1_v, x2_v, y2_v,
               ar_v, outbuf_v, sem):
    wid = lax.axis_index("s") * _NC + lax.axis_index("c")
    lane = lax.iota(jnp.int32, 16)
    zero16 = jnp.zeros((16,), jnp.int32)

    def zero_hist(b, _):
        for u in range(4):
            hist_v[pl.ds((b * 4 + u) * 16, 16)] = zero16
        return 0

    lax.fori_loop(0, 64, zero_hist, 0)

    args = (conf_hbm, feat_hbm, out_hbm, scores_v, keys_v, hist_v, ring_v,
            cand_v, idx_v, feats_v, sc_v, x1_v, y1_v, x2_v, y2_v, ar_v,
            outbuf_v, sem, lane)
    _do_task(wid, *args)

    @pl.when(wid + _NW < _TASKS)
    def _():
        _do_task(wid + _NW, *args)


def kernel(loc_data, conf_data, prior_data):
    num = loc_data.shape[0]
    conf_rows = jnp.transpose(conf_data[:, :, 1:], (0, 2, 1)).reshape(
        num * (_NUM_CLASSES - 1), _N)
    feat = jnp.concatenate(
        [loc_data.reshape(num * _N, 4),
         jnp.broadcast_to(prior_data, (num, _N, 4)).reshape(num * _N, 4)],
        axis=1)
    out40 = _sc_detect(conf_rows, feat)
    out = out40.reshape(num, _NUM_CLASSES - 1, _TOP_K, 5)
    zeros0 = jnp.zeros((num, 1, _TOP_K, 5), jnp.float32)
    return jnp.concatenate([zeros0, out], axis=1)


# speedup vs baseline: 632.0375x; 1.3228x over previous
import functools

import jax
import jax.numpy as jnp
from jax import lax
from jax.experimental import pallas as pl
from jax.experimental.pallas import tpu as pltpu
from jax.experimental.pallas import tpu_sc as plsc

_NUM_CLASSES = 21
_TOP_K = 200
_CONF = 0.01
_NMS_T = 0.45
_N = 20000
_NV = _N // 16
_NC, _NS = 2, 16
_NW = _NC * _NS
_TASKS = 40
_CAND = 208
_CV = _CAND // 16
_UNROLL = 5


def _do_task(task, conf_hbm, feat_hbm, out_hbm, scores_v, keys_v, hist_v,
             ring_v, cand_v, idx_v, feats_v, sc_v, x1_v, y1_v, x2_v,
             y2_v, ar_v, outbuf_v, clk_v, cli_v, sem, lane):
    ones16 = jnp.ones((16,), jnp.int32)
    zero16 = jnp.zeros((16,), jnp.int32)
    neg_inf = jnp.full((16,), -jnp.inf, jnp.float32)

    pltpu.sync_copy(conf_hbm.at[task], scores_v)

    def hist_body(j, _):
        for u in range(_UNROLL):
            base = pl.ds((j * _UNROLL + u) * 16, 16)
            s = scores_v[base]
            valid = s > _CONF
            keys_v[base] = jnp.where(valid, plsc.bitcast(s, jnp.int32) + 1, 0)
            bucket = (lax.shift_right_logical(
                plsc.bitcast(s + 1.0, jnp.int32), 15) & 0xFF)
            plsc.addupdate_scatter(hist_v, [bucket * 16 + lane], ones16,
                                   mask=valid)
        return 0

    lax.fori_loop(0, _NV // _UNROLL, hist_body, 0)

    def scan_body(k, carry):
        cum, bsel, cnt_above = carry
        b = 255 - k
        row = pl.ds(b * 16, 16)
        c = jnp.sum(hist_v[row])
        hist_v[row] = zero16
        hit = (bsel < 0) & (cum + c >= _TOP_K)
        bsel = jnp.where(hit, b, bsel)
        cnt_above = jnp.where(hit, cum, cnt_above)
        return cum + c, bsel, cnt_above

    _, bsel, _ = lax.fori_loop(
        0, 256, scan_body, (jnp.int32(0), jnp.int32(-1), jnp.int32(0)))

    for c in range(_CV):
        cand_v[pl.ds(c * 16, 16)] = zero16

    def comp_body(j, carry):
        gt_base, cl_base = carry
        for u in range(_UNROLL):
            jj = j * _UNROLL + u
            base = pl.ds(jj * 16, 16)
            key = keys_v[base]
            s = scores_v[base]
            bucket = (lax.shift_right_logical(
                plsc.bitcast(s + 1.0, jnp.int32), 15) & 0xFF)
            bucket = jnp.where(key > 0, bucket, -1)
            gidx = jj * 16 + lane
            m_gt = bucket > bsel
            pos = jnp.maximum(
                gt_base + plsc.cumsum(jnp.where(m_gt, 1, 0)) - 1, 0)
            plsc.store_scatter(cand_v, [pos], gidx, mask=m_gt)
            gt_base = gt_base + plsc.all_reduce_population_count(m_gt)
            m_cl = bucket == bsel
            cpos = jnp.maximum(
                cl_base + plsc.cumsum(jnp.where(m_cl, 1, 0)) - 1, 0)
            plsc.store_scatter(clk_v, [cpos], key, mask=m_cl)
            plsc.store_scatter(cli_v, [cpos], gidx, mask=m_cl)
            cl_base = cl_base + plsc.all_reduce_population_count(m_cl)
        return gt_base, cl_base

    gt_base, cl_base = lax.fori_loop(0, _NV // _UNROLL, comp_body,
                                     (zero16, zero16))
    n_gt0 = jnp.max(gt_base)
    n_cl = jnp.max(cl_base)
    nvc = lax.shift_right_logical(n_cl + 15, 4)

    needb = _TOP_K - n_gt0

    lo = jnp.int32(0)
    hi = jnp.int32(0x3F800002)
    for _ in range(30):
        mid = lax.shift_right_logical(lo + hi, 1)

        def cnt_body(v, acc, mid=mid):
            k = clk_v[pl.ds(v * 16, 16)]
            lm = (v * 16 + lane) < n_cl
            return acc + jnp.where(lm & (k >= mid), 1, 0)

        cnt = jnp.sum(lax.fori_loop(0, nvc, cnt_body, zero16))
        ge = cnt >= needb
        lo = jnp.where(ge, mid, lo)
        hi = jnp.where(ge, hi, mid)
    kth = lo

    def ccomp_body(v, carry):
        g2, eq_base = carry
        k = clk_v[pl.ds(v * 16, 16)]
        gi = cli_v[pl.ds(v * 16, 16)]
        lm = (v * 16 + lane) < n_cl
        m_gt = lm & (k > kth)
        pos = jnp.maximum(g2 + plsc.cumsum(jnp.where(m_gt, 1, 0)) - 1, 0)
        plsc.store_scatter(cand_v, [pos], gi, mask=m_gt)
        g2 = g2 + plsc.all_reduce_population_count(m_gt)
        m_eq = lm & (k == kth)
        pos_e = (eq_base + plsc.cumsum(jnp.where(m_eq, 1, 0)) - 1) & 255
        plsc.store_scatter(ring_v, [pos_e], gi, mask=m_eq)
        eq_base = eq_base + plsc.all_reduce_population_count(m_eq)
        return g2, eq_base

    gt2, eq_base = lax.fori_loop(0, nvc, ccomp_body, (gt_base, zero16))
    n_gt = jnp.max(gt2)
    n_eq = jnp.max(eq_base)
    need4 = _TOP_K - n_gt
    for c in range(_CV):
        offs = c * 16 + lane
        selm = offs < need4
        src = (n_eq - need4 + offs) & 255
        val = plsc.load_gather(ring_v, [src], mask=selm)
        dst = jnp.minimum(n_gt + offs, _CAND - 1)
        plsc.store_scatter(cand_v, [dst], val, mask=selm)

    ioff = jnp.where(task >= 20, _N, 0)
    ms = []
    for c in range(_CV):
        base = pl.ds(c * 16, 16)
        slot = c * 16 + lane
        ci = cand_v[base]
        valid_slot = slot < _TOP_K
        ci = jnp.where(valid_slot, ci, 0)
        scv = plsc.load_gather(scores_v, [ci])
        active = valid_slot & (scv > _CONF)
        sc_v[base] = scv
        ms.append(jnp.where(active, scv, neg_inf))
        idx_v[base] = ci + ioff

    pltpu.async_copy(feat_hbm.at[idx_v.at[pl.ds(0, 104)]],
                     feats_v.at[pl.ds(0, 104), :], sem).wait()
    pltpu.async_copy(feat_hbm.at[idx_v.at[pl.ds(104, 104)]],
                     feats_v.at[pl.ds(104, 104), :], sem).wait()

    for c in range(_CV):
        base = pl.ds(c * 16, 16)
        row = c * 16 + lane

        def col(k):
            return plsc.load_gather(feats_v,
                                    [row, jnp.full((16,), k, jnp.int32)])

        lx, ly, lw, lh = col(0), col(1), col(2), col(3)
        px, py, pw, ph = col(4), col(5), col(6), col(7)
        x = px + (lx * 0.1) * pw
        y = py + (ly * 0.1) * ph
        w = pw * jnp.exp(lw * 0.2)
        h = ph * jnp.exp(lh * 0.2)
        x1 = x - w * 0.5
        y1 = y - h * 0.5
        x2 = w + x1
        y2 = h + y1
        x1_v[base] = x1
        y1_v[base] = y1
        x2_v[base] = x2
        y2_v[base] = y2
        ar_v[base] = (x2 - x1) * (y2 - y1)

    bestval = neg_inf
    bestslot = jnp.full((16,), -1, jnp.int32)
    for c in range(_CV):
        slotv = c * 16 + lane
        upd = ms[c] >= bestval
        bestslot = jnp.where(upd, slotv, bestslot)
        bestval = jnp.maximum(bestval, ms[c])
    m0 = jnp.max(bestval)
    sel0 = jnp.maximum(
        jnp.max(jnp.where(bestval == m0, bestslot, -1)), 0)

    def nms_body(t, carry):
        m = carry[0]
        sel = carry[1]
        ms = list(carry[2:])
        any_active = m > -jnp.inf
        sels = zero16 + sel
        sx1 = plsc.load_gather(x1_v, [sels])
        sy1 = plsc.load_gather(y1_v, [sels])
        sx2 = plsc.load_gather(x2_v, [sels])
        sy2 = plsc.load_gather(y2_v, [sels])
        sar = plsc.load_gather(ar_v, [sels])
        ssc = plsc.load_gather(sc_v, [sels])

        bestval = neg_inf
        bestslot = jnp.full((16,), -1, jnp.int32)
        new_ms = []
        for c in range(_CV):
            base = pl.ds(c * 16, 16)
            slotv = c * 16 + lane
            xx1 = jnp.maximum(x1_v[base], sx1)
            yy1 = jnp.maximum(y1_v[base], sy1)
            xx2 = jnp.minimum(x2_v[base], sx2)
            yy2 = jnp.minimum(y2_v[base], sy2)
            inter = (jnp.maximum(xx2 - xx1, 0.0)
                     * jnp.maximum(yy2 - yy1, 0.0))
            union = ar_v[base] + sar - inter
            iou = inter / union
            nm = jnp.where(iou <= _NMS_T, ms[c], neg_inf)
            nm = jnp.where(slotv == sel, neg_inf, nm)
            nm = jnp.where(any_active, nm, ms[c])
            upd = nm >= bestval
            bestslot = jnp.where(upd, slotv, bestslot)
            bestval = jnp.maximum(bestval, nm)
            new_ms.append(nm)
        m_next = jnp.max(bestval)
        sel_next = jnp.maximum(
            jnp.max(jnp.where(bestval == m_next, bestslot, -1)), 0)

        ovals = jnp.where(lane == 0, ssc,
                          jnp.where(lane == 1, sx1,
                                    jnp.where(lane == 2, sy1,
                                              jnp.where(lane == 3, sx2,
                                                        sy2))))
        ovals = jnp.where(any_active, ovals, 0.0)
        plsc.store_scatter(outbuf_v, [t * 5 + lane], ovals, mask=lane < 5)
        return (m_next, sel_next, *new_ms)

    lax.fori_loop(0, _TOP_K, nms_body, (m0, sel0, *ms))

    pltpu.sync_copy(outbuf_v, out_hbm.at[task])


@functools.partial(
    pl.kernel,
    out_type=jax.ShapeDtypeStruct((_TASKS, _TOP_K * 5), jnp.float32),
    mesh=plsc.VectorSubcoreMesh(core_axis_name="c", subcore_axis_name="s"),
    scratch_types=[
        pltpu.VMEM((_N,), jnp.float32),
        pltpu.VMEM((_N,), jnp.int32),
        pltpu.VMEM((4096,), jnp.int32),
        pltpu.VMEM((256,), jnp.int32),
        pltpu.VMEM((_CAND,), jnp.int32),
        pltpu.VMEM((_CAND,), jnp.int32),
        pltpu.VMEM((_CAND, 8), jnp.float32),
        pltpu.VMEM((_CAND,), jnp.float32),
        pltpu.VMEM((_CAND,), jnp.float32),
        pltpu.VMEM((_CAND,), jnp.float32),
        pltpu.VMEM((_CAND,), jnp.float32),
        pltpu.VMEM((_CAND,), jnp.float32),
        pltpu.VMEM((_CAND,), jnp.float32),
        pltpu.VMEM((_TOP_K * 5,), jnp.float32),
        pltpu.VMEM((_N,), jnp.int32),
        pltpu.VMEM((_N,), jnp.int32),
        pltpu.SemaphoreType.DMA,
    ],
    compiler_params=pltpu.CompilerParams(needs_layout_passes=False,
                                         use_tc_tiling_on_sc=False),
)
def _sc_detect(conf_hbm, feat_hbm, out_hbm, scores_v, keys_v, hist_v, ring_v,
               cand_v, idx_v, feats_v, sc_v, x1_v, y1_v, x2_v, y2_v,
               ar_v, outbuf_v, clk_v, cli_v, sem):
    wid = lax.axis_index("s") * _NC + lax.axis_index("c")
    lane = lax.iota(jnp.int32, 16)
    zero16 = jnp.zeros((16,), jnp.int32)

    def zero_hist(b, _):
        for u in range(4):
            hist_v[pl.ds((b * 4 + u) * 16, 16)] = zero16
        return 0

    lax.fori_loop(0, 64, zero_hist, 0)

    args = (conf_hbm, feat_hbm, out_hbm, scores_v, keys_v, hist_v, ring_v,
            cand_v, idx_v, feats_v, sc_v, x1_v, y1_v, x2_v, y2_v, ar_v,
            outbuf_v, clk_v, cli_v, sem, lane)
    _do_task(wid, *args)

    @pl.when(wid + _NW < _TASKS)
    def _():
        _do_task(wid + _NW, *args)


def kernel(loc_data, conf_data, prior_data):
    num = loc_data.shape[0]
    conf_rows = jnp.transpose(conf_data[:, :, 1:], (0, 2, 1)).reshape(
        num * (_NUM_CLASSES - 1), _N)
    feat = jnp.concatenate(
        [loc_data.reshape(num * _N, 4),
         jnp.broadcast_to(prior_data, (num, _N, 4)).reshape(num * _N, 4)],
        axis=1)
    out40 = _sc_detect(conf_rows, feat)
    out = out40.reshape(num, _NUM_CLASSES - 1, _TOP_K, 5)
    zeros0 = jnp.zeros((num, 1, _TOP_K, 5), jnp.float32)
    return jnp.concatenate([zeros0, out], axis=1)
